# Initial kernel scaffold; baseline (speedup 1.0000x reference)
import jax
import jax.numpy as jnp
from jax.experimental import pallas as pl


def kernel(X, adj_indices, adj_values, W, b):
    raise NotImplementedError("write your pallas kernel here")


# speedup vs baseline: 11.4707x; 11.4707x over previous
import functools

import jax
import jax.numpy as jnp
from jax import lax
from jax.experimental import pallas as pl
from jax.experimental.pallas import tpu as pltpu
from jax.experimental.pallas import tpu_sc as plsc

NC = 2
NS = 16
LANES = 16
NBUF = 4


def _linear_body(x_ref, w_ref, b_ref, o_ref):
    o_ref[...] = lax.dot_general(
        x_ref[...], w_ref[...], (((1,), (1,)), ((), ())),
        preferred_element_type=jnp.float32) + b_ref[...]


def _combine_body(p_ref, o_ref):
    o_ref[...] = p_ref[0] + p_ref[1]


def _make_scatter(n, d, chunks, k):
    rps = n // NS
    mesh = plsc.VectorSubcoreMesh(
        core_axis_name="c", subcore_axis_name="s",
        num_cores=NC, num_subcores=NS)

    @functools.partial(
        pl.kernel,
        out_type=jax.ShapeDtypeStruct((NC, n, d), jnp.float32),
        mesh=mesh,
        scratch_types=[
            pltpu.VMEM_SHARED((n, d), jnp.float32),
            pltpu.VMEM((chunks, k), jnp.int32),
            pltpu.VMEM((chunks, k), jnp.int32),
            pltpu.VMEM((chunks, k), jnp.float32),
            pltpu.VMEM((NBUF, k, d), jnp.float32),
            pltpu.SemaphoreType.DMA,
            pltpu.SemaphoreType.DMA,
            pltpu.SemaphoreType.DMA,
        ],
        compiler_params=pltpu.CompilerParams(
            use_tc_tiling_on_sc=False, needs_layout_passes=False),
    )
    def scatter(h, colr, rowr, valr, zeros, out, acc, colv, rowv, valv,
                msg, sem_g, sem_s, sem_in):
        cid = lax.axis_index("c")
        sid = lax.axis_index("s")
        wid = cid * NS + sid

        pltpu.async_copy(colr.at[wid], colv, sem_in)
        pltpu.async_copy(rowr.at[wid], rowv, sem_in)
        pltpu.async_copy(valr.at[wid], valv, sem_in)
        pltpu.sync_copy(zeros.at[pl.ds(sid * rps, rps)],
                        acc.at[pl.ds(sid * rps, rps)])
        pltpu.make_async_copy(colr.at[wid], colv, sem_in).wait()
        pltpu.make_async_copy(rowr.at[wid], rowv, sem_in).wait()
        pltpu.make_async_copy(valr.at[wid], valv, sem_in).wait()
        plsc.subcore_barrier()

        def gather_wait():
            pltpu.make_async_copy(h.at[colv.at[0]], msg.at[0], sem_g).wait()

        def scatter_wait():
            pltpu.make_async_copy(msg.at[0], acc.at[rowv.at[0]],
                                  sem_s).wait()

        pltpu.async_copy(h.at[colv.at[0]], msg.at[0], sem_g)
        pltpu.async_copy(h.at[colv.at[1]], msg.at[1], sem_g)

        def chunk_body(j, carry):
            b = lax.rem(j, NBUF)
            bn = lax.rem(j + 2, NBUF)
            @pl.when(j >= 2)
            def _():
                scatter_wait()

            @pl.when(j + 2 < chunks)
            def _():
                pltpu.async_copy(h.at[colv.at[j + 2]], msg.at[bn], sem_g)

            gather_wait()
            mb = msg.at[b]
            vj = jnp.full((LANES,), j, jnp.int32)
            for i in range(k):
                vv = plsc.load_gather(
                    valv, [vj, jnp.full((LANES,), i, jnp.int32)])
                for f in range(d // LANES):
                    sl = pl.ds(f * LANES, LANES)
                    mb[i, sl] = mb[i, sl] * vv
            pltpu.async_copy(mb, acc.at[rowv.at[j]], sem_s, add=True)
            return carry

        lax.fori_loop(0, chunks, chunk_body, 0)
        scatter_wait()
        scatter_wait()

        plsc.subcore_barrier()
        pltpu.sync_copy(acc.at[pl.ds(sid * rps, rps)],
                        out.at[cid, pl.ds(sid * rps, rps)])

    return scatter


@jax.jit
def kernel(X, adj_indices, adj_values, W, b):
    n, d_in = X.shape
    d_out = W.shape[0]
    e = adj_values.shape[0]
    nw = NC * NS
    k = 40
    epw = -(-e // (nw * k)) * k
    chunks = epw // k
    ep = epw * nw
    pad = ep - e

    row_blocks = 10
    rb = n // row_blocks
    h = pl.pallas_call(
        _linear_body,
        grid=(row_blocks,),
        in_specs=[
            pl.BlockSpec((rb, d_in), lambda i: (i, 0)),
            pl.BlockSpec((d_out, d_in), lambda i: (0, 0)),
            pl.BlockSpec((1, d_out), lambda i: (0, 0)),
        ],
        out_specs=pl.BlockSpec((rb, d_out), lambda i: (i, 0)),
        out_shape=jax.ShapeDtypeStruct((n, d_out), jnp.float32),
    )(X, W, b.reshape(1, d_out))

    colr = jnp.pad(adj_indices[1], (0, pad)).reshape(nw, chunks, k)
    rowr = jnp.pad(adj_indices[0], (0, pad)).reshape(nw, chunks, k)
    valr = jnp.pad(adj_values, (0, pad)).reshape(nw, chunks, k)
    zeros = jnp.zeros((n, d_out), jnp.float32)

    partials = _make_scatter(n, d_out, chunks, k)(h, colr, rowr, valr, zeros)

    out = pl.pallas_call(
        _combine_body,
        grid=(row_blocks,),
        in_specs=[pl.BlockSpec((NC, rb, d_out), lambda i: (0, i, 0))],
        out_specs=pl.BlockSpec((rb, d_out), lambda i: (i, 0)),
        out_shape=jax.ShapeDtypeStruct((n, d_out), jnp.float32),
    )(partials)
    return out
